# baseline (speedup 1.0000x reference)
import numpy as np
import jax
import jax.numpy as jnp
from jax.experimental import pallas as pl
from jax.experimental.pallas import tpu as pltpu

DIM = 192
WS = 7
KNN = 15
PW = 32
NPTS = PW * PW
C8 = DIM // 8
Wd_ = WS * PW


def _grid_const():
    gi, gj = np.meshgrid(np.arange(PW, dtype=np.float32),
                         np.arange(PW, dtype=np.float32), indexing="ij")
    grid = np.stack([gi, gj], axis=-1).reshape(NPTS, 2)
    mean = grid.mean(0)
    std = grid.std(0, ddof=1)
    return ((grid - mean) / (std + 1e-5)).astype(np.float32)


_GRID2 = _grid_const()


def _body(ab_ref, x_ref, wf_ref, bf_ref, grid_ref, out_ref, xt_scr, ot_scr):
    hg = pl.program_id(1)
    alpha = ab_ref[0]
    beta = ab_ref[1]

    def _stage_in(j):
        t = x_ref[:, 0, :, j * PW:(j + 1) * PW]
        xt_scr[j * NPTS:(j + 1) * NPTS, :] = jnp.transpose(
            t.reshape(DIM, NPTS), (1, 0))

    @pl.when(hg == 0)
    def _():
        _stage_in(0)

    for _j in range(1, WS):
        @pl.when(hg == _j - 1)
        def _(j=_j):
            _stage_in(j)

    off = pl.multiple_of(hg * NPTS, NPTS)
    x = xt_scr[pl.ds(off, NPTS), :]
    f = jax.lax.dot_general(x, wf_ref[...], (((1,), (0,)), ((), ())),
                            preferred_element_type=jnp.float32)
    f = f + bf_ref[...]
    aug = jnp.concatenate([f, grid_ref[...]], axis=1)
    nrm = jnp.maximum(jnp.sqrt(jnp.sum(aug * aug, axis=1, keepdims=True)), 1e-8)
    xn = aug / nrm
    s = jax.lax.dot_general(xn, xn, (((1,), (1,)), ((), ())),
                            preferred_element_type=jnp.float32)

    def _edge_e(v):
        return jnp.exp(1.0 / (1.0 + jnp.exp(-(beta + alpha * v))))

    efull = _edge_e(s)
    m = jnp.max(s, axis=1, keepdims=True)
    den = _edge_e(m)
    for _ in range(KNN - 1):
        m = jnp.max(jnp.where(s < m, s, -3.0e38), axis=1, keepdims=True)
        den = den + _edge_e(m)
    p = jnp.where(s >= m, efull, 0.0) / den
    o = jax.lax.dot_general(p, x, (((1,), (0,)), ((), ())),
                            preferred_element_type=jnp.float32)
    ot_scr[pl.ds(off, NPTS), :] = o

    def _stage_out(j):
        oj = jnp.transpose(ot_scr[j * NPTS:(j + 1) * NPTS, :], (1, 0))
        out_ref[:, 0, :, j * PW:(j + 1) * PW] = oj.reshape(DIM, PW, PW)

    for _j in range(WS - 1):
        @pl.when(hg == _j + 1)
        def _(j=_j):
            _stage_out(j)

    @pl.when(hg == WS - 1)
    def _():
        _stage_out(WS - 1)


def kernel(x_in, Wf, bf, edge_alpha, edge_beta):
    B, C, H, Wd = x_in.shape
    ab = jnp.stack([edge_alpha[0], edge_beta[0]])
    bf2 = bf.reshape(1, C8)
    grid2 = jnp.asarray(_GRID2)
    xs = x_in.reshape(DIM, WS, PW, Wd)
    out = pl.pallas_call(
        _body,
        grid=(WS, WS),
        in_specs=[
            pl.BlockSpec(memory_space=pltpu.SMEM),
            pl.BlockSpec((DIM, 1, PW, Wd), lambda i, j: (0, i, 0, 0)),
            pl.BlockSpec((DIM, C8), lambda i, j: (0, 0)),
            pl.BlockSpec((1, C8), lambda i, j: (0, 0)),
            pl.BlockSpec((NPTS, 2), lambda i, j: (0, 0)),
        ],
        out_specs=pl.BlockSpec((DIM, 1, PW, Wd), lambda i, j: (0, i, 0, 0)),
        out_shape=jax.ShapeDtypeStruct((DIM, WS, PW, Wd), jnp.float32),
        scratch_shapes=[
            pltpu.VMEM((WS * NPTS, DIM), jnp.float32),
            pltpu.VMEM((WS * NPTS, DIM), jnp.float32),
        ],
    )(ab, xs, Wf.T, bf2, grid2)
    return out.reshape(B, C, H, Wd)


# speedup vs baseline: 1.0703x; 1.0703x over previous
import numpy as np
import jax
import jax.numpy as jnp
from jax.experimental import pallas as pl
from jax.experimental.pallas import tpu as pltpu

DIM = 192
WS = 7
KNN = 15
PW = 32
NPTS = PW * PW
C8 = DIM // 8


def _grid_const():
    gi, gj = np.meshgrid(np.arange(PW, dtype=np.float32),
                         np.arange(PW, dtype=np.float32), indexing="ij")
    grid = np.stack([gi, gj], axis=-1).reshape(NPTS, 2)
    mean = grid.mean(0)
    std = grid.std(0, ddof=1)
    return ((grid - mean) / (std + 1e-5)).astype(np.float32)


_GRID2 = _grid_const()


def _body(ab_ref, x_ref, wf_ref, bf_ref, grid_ref, out_ref, xt_scr, ot_scr):
    hg = pl.program_id(1)
    alpha = ab_ref[0]
    beta = ab_ref[1]

    @pl.when(hg == 0)
    def _():
        for j in range(WS):
            t = x_ref[:, 0, :, j * PW:(j + 1) * PW]
            xt_scr[:, j * NPTS:(j + 1) * NPTS] = t.reshape(DIM, NPTS)

    off = pl.multiple_of(hg * NPTS, NPTS)
    x = xt_scr[:, pl.ds(off, NPTS)]
    f = jax.lax.dot_general(x, wf_ref[...], (((0,), (1,)), ((), ())),
                            preferred_element_type=jnp.float32)
    f = f + bf_ref[...]
    aug = jnp.concatenate([f, grid_ref[...]], axis=1)
    nrm = jnp.maximum(jnp.sqrt(jnp.sum(aug * aug, axis=1, keepdims=True)), 1e-8)
    xn = aug / nrm
    s = jax.lax.dot_general(xn, xn, (((1,), (1,)), ((), ())),
                            preferred_element_type=jnp.float32)

    def _edge_e(v):
        return jnp.exp(1.0 / (1.0 + jnp.exp(-(beta + alpha * v))))

    m = jnp.max(s, axis=1, keepdims=True)
    den = _edge_e(m)
    for _ in range(KNN - 1):
        m = jnp.max(jnp.where(s < m, s, -3.0e38), axis=1, keepdims=True)
        den = den + _edge_e(m)
    p = jnp.where(s >= m, _edge_e(s), 0.0) / den
    o = jax.lax.dot_general(x, p, (((1,), (1,)), ((), ())),
                            preferred_element_type=jnp.float32)
    ot_scr[:, pl.ds(off, NPTS)] = o

    @pl.when(hg == WS - 1)
    def _():
        for j in range(WS):
            oj = ot_scr[:, j * NPTS:(j + 1) * NPTS].reshape(DIM, PW, PW)
            out_ref[:, 0, :, j * PW:(j + 1) * PW] = oj


def kernel(x_in, Wf, bf, edge_alpha, edge_beta):
    B, C, H, Wd = x_in.shape
    ab = jnp.stack([edge_alpha[0], edge_beta[0]])
    bf2 = bf.reshape(1, C8)
    grid2 = jnp.asarray(_GRID2)
    xs = x_in.reshape(DIM, WS, PW, Wd)
    out = pl.pallas_call(
        _body,
        grid=(WS, WS),
        in_specs=[
            pl.BlockSpec(memory_space=pltpu.SMEM),
            pl.BlockSpec((DIM, 1, PW, Wd), lambda i, j: (0, i, 0, 0)),
            pl.BlockSpec((C8, DIM), lambda i, j: (0, 0)),
            pl.BlockSpec((1, C8), lambda i, j: (0, 0)),
            pl.BlockSpec((NPTS, 2), lambda i, j: (0, 0)),
        ],
        out_specs=pl.BlockSpec((DIM, 1, PW, Wd), lambda i, j: (0, i, 0, 0)),
        out_shape=jax.ShapeDtypeStruct((DIM, WS, PW, Wd), jnp.float32),
        scratch_shapes=[
            pltpu.VMEM((DIM, WS * NPTS), jnp.float32),
            pltpu.VMEM((DIM, WS * NPTS), jnp.float32),
        ],
    )(ab, xs, Wf, bf2, grid2)
    return out.reshape(B, C, H, Wd)
